# baseline (speedup 1.0000x reference)
import functools

import jax
import jax.numpy as jnp
from jax import lax
from jax.experimental import pallas as pl
from jax.experimental.pallas import tpu as pltpu
from jax.experimental.pallas import tpu_sc as plsc

ROWS = 64
N = 32768
LANES = 16
NV = N // LANES
NBINS = 2048
SHIFTS = (0, 11, 22)
NW = 32
ROWS_PER_W = ROWS // NW
CH = 8
VPC = NV // CH


def _to_key(vf):
  u = plsc.bitcast(vf, jnp.int32)
  m = lax.shift_right_arithmetic(u, 31)
  flip = lax.bitwise_not(lax.bitwise_or(m, jnp.int32(-(2**31))))
  return lax.bitwise_xor(u, flip)


def _digit(k, shift):
  return lax.bitwise_and(
      lax.shift_right_logical(k, jnp.int32(shift)), jnp.int32(NBINS - 1)
  )


@functools.cache
def _build():
  mesh = plsc.VectorSubcoreMesh(core_axis_name="c", subcore_axis_name="s")

  @functools.partial(
      pl.kernel,
      out_type=jax.ShapeDtypeStruct((ROWS, N), jnp.float32),
      mesh=mesh,
      compiler_params=pltpu.CompilerParams(needs_layout_passes=False),
      scratch_types=[
          pltpu.VMEM((N,), jnp.float32),
          pltpu.VMEM((N,), jnp.float32),
          pltpu.VMEM((N,), jnp.float32),
          pltpu.VMEM((CH * NBINS,), jnp.int32),
      ],
  )
  def ranker(seq_hbm, out_hbm, key_ref, bufa, bufb, hist):
    wid = lax.axis_index("s") * 2 + lax.axis_index("c")

    def load_keyed(src, shift, j, i, transform):
      v_base = (j * VPC + i) * LANES
      sl = pl.ds(v_base, LANES)
      if src is None:
        if transform:
          k = _to_key(key_ref[sl])
          key_ref[sl] = plsc.bitcast(k, jnp.float32)
        else:
          k = plsc.bitcast(key_ref[sl], jnp.int32)
        srci = lax.iota(jnp.int32, LANES) + v_base
      else:
        srci = plsc.bitcast(src[sl], jnp.int32)
        k = plsc.bitcast(plsc.load_gather(key_ref, [srci]), jnp.int32)
      d = _digit(k, shift) + jnp.int32(j * NBINS)
      return d, srci

    def run_pass(shift, src, dst, final):
      @pl.loop(0, CH * NBINS // LANES)
      def _clear(i):
        hist[pl.ds(i * LANES, LANES)] = jnp.zeros((LANES,), jnp.int32)

      @pl.loop(0, VPC)
      def _histogram(i):
        for j in range(CH):
          d, _ = load_keyed(src, shift, j, i, transform=src is None and shift == SHIFTS[0])
          counts, last = plsc.scan_count(d)
          plsc.addupdate_scatter(hist, [d], counts, mask=last)

      @pl.loop(0, NBINS // LANES, init_carry=jnp.int32(0))
      def _prefix(i, carry):
        hs = [hist[pl.ds(j * NBINS + i * LANES, LANES)] for j in range(CH)]
        total = functools.reduce(lax.add, hs)
        c = plsc.cumsum(total)
        run = c - total + carry
        for j in range(CH):
          hist[pl.ds(j * NBINS + i * LANES, LANES)] = run
          if j < CH - 1:
            run = run + hs[j]
        return carry + jnp.sum(total)

      @pl.loop(0, VPC)
      def _place(i):
        for j in range(CH):
          d, srci = load_keyed(src, shift, j, i, transform=False)
          counts, last = plsc.scan_count(d)
          base = plsc.load_gather(hist, [d])
          pos = base + counts - jnp.int32(1)
          if final:
            val = (pos + 1).astype(jnp.float32) * jnp.float32(1.0 / N)
            plsc.store_scatter(dst, [srci], val)
          else:
            plsc.store_scatter(dst, [pos], plsc.bitcast(srci, jnp.float32))
          plsc.addupdate_scatter(hist, [d], counts, mask=last)

    for r in range(ROWS_PER_W):
      row = wid * ROWS_PER_W + r
      pltpu.sync_copy(seq_hbm.at[row], key_ref)
      run_pass(SHIFTS[0], None, bufa, False)
      run_pass(SHIFTS[1], bufa, bufb, False)
      run_pass(SHIFTS[2], bufb, bufa, True)
      pltpu.sync_copy(bufa, out_hbm.at[row])

  return ranker


def kernel(sequence):
  return _build()(sequence)


# speedup vs baseline: 1.0782x; 1.0782x over previous
import functools

import jax
import jax.numpy as jnp
from jax import lax
from jax.experimental import pallas as pl
from jax.experimental.pallas import tpu as pltpu
from jax.experimental.pallas import tpu_sc as plsc

ROWS = 64
N = 32768
LANES = 16
NV = N // LANES
NBINS = 2048
SHIFTS = (0, 11, 22)
NW = 32
ROWS_PER_W = ROWS // NW
CH = 8
VPC = NV // CH


def _to_key(vf):
  u = plsc.bitcast(vf, jnp.int32)
  m = lax.shift_right_arithmetic(u, 31)
  flip = lax.bitwise_not(lax.bitwise_or(m, jnp.int32(-(2**31))))
  return lax.bitwise_xor(u, flip)


def _digit(k, shift):
  return lax.bitwise_and(
      lax.shift_right_logical(k, jnp.int32(shift)), jnp.int32(NBINS - 1)
  )


@functools.cache
def _build():
  mesh = plsc.VectorSubcoreMesh(core_axis_name="c", subcore_axis_name="s")

  @functools.partial(
      pl.kernel,
      out_type=jax.ShapeDtypeStruct((ROWS, N), jnp.float32),
      mesh=mesh,
      compiler_params=pltpu.CompilerParams(needs_layout_passes=False),
      scratch_types=[
          pltpu.VMEM((N,), jnp.float32),
          pltpu.VMEM((N,), jnp.float32),
          pltpu.VMEM((N,), jnp.float32),
      ] + [pltpu.VMEM((NBINS,), jnp.int32) for _ in range(CH)],
  )
  def ranker(seq_hbm, out_hbm, key_ref, bufa, bufb, *hists):
    wid = lax.axis_index("s") * 2 + lax.axis_index("c")

    def load_keyed(src, shift, j, i, transform):
      v_base = (j * VPC + i) * LANES
      sl = pl.ds(v_base, LANES)
      if src is None:
        if transform:
          k = _to_key(key_ref[sl])
          key_ref[sl] = plsc.bitcast(k, jnp.float32)
        else:
          k = plsc.bitcast(key_ref[sl], jnp.int32)
        srci = lax.iota(jnp.int32, LANES) + v_base
      else:
        srci = plsc.bitcast(src[sl], jnp.int32)
        k = plsc.bitcast(plsc.load_gather(key_ref, [srci]), jnp.int32)
      return _digit(k, shift), srci

    def run_pass(shift, src, dst, final):
      @pl.loop(0, NBINS // LANES)
      def _clear(i):
        sl = pl.ds(i * LANES, LANES)
        for j in range(CH):
          hists[j][sl] = jnp.zeros((LANES,), jnp.int32)

      @pl.loop(0, VPC)
      def _histogram(i):
        for j in range(CH):
          d, _ = load_keyed(
              src, shift, j, i, transform=src is None and shift == SHIFTS[0]
          )
          counts, last = plsc.scan_count(d)
          plsc.addupdate_scatter(hists[j], [d], counts, mask=last)

      @pl.loop(0, NBINS // LANES, init_carry=jnp.int32(0))
      def _prefix(i, carry):
        sl = pl.ds(i * LANES, LANES)
        hs = [hists[j][sl] for j in range(CH)]
        total = functools.reduce(lax.add, hs)
        c = plsc.cumsum(total)
        run = c - total + carry
        for j in range(CH):
          hists[j][sl] = run
          if j < CH - 1:
            run = run + hs[j]
        return carry + jnp.sum(total)

      @pl.loop(0, VPC)
      def _place(i):
        for j in range(CH):
          d, srci = load_keyed(src, shift, j, i, transform=False)
          counts, last = plsc.scan_count(d)
          base = plsc.load_gather(hists[j], [d])
          pos = base + counts - jnp.int32(1)
          if final:
            val = (pos + 1).astype(jnp.float32) * jnp.float32(1.0 / N)
            plsc.store_scatter(dst, [srci], val)
          else:
            plsc.store_scatter(dst, [pos], plsc.bitcast(srci, jnp.float32))
          plsc.addupdate_scatter(hists[j], [d], counts, mask=last)

    for r in range(ROWS_PER_W):
      row = wid * ROWS_PER_W + r
      pltpu.sync_copy(seq_hbm.at[row], key_ref)
      run_pass(SHIFTS[0], None, bufa, False)
      run_pass(SHIFTS[1], bufa, bufb, False)
      run_pass(SHIFTS[2], bufb, bufa, True)
      pltpu.sync_copy(bufa, out_hbm.at[row])

  return ranker


def kernel(sequence):
  return _build()(sequence)
